# baseline (speedup 1.0000x reference)
import jax
import jax.numpy as jnp
from jax.experimental import pallas as pl
from jax.experimental.pallas import tpu as pltpu

_K_BY_HEAD = (10, 20, 40, 500)
_ROWS_PER_BLOCK = 256
_MAX_ITERS = 72


def _monotone_i32(b):
    return jnp.where(b >= 0, b, b ^ jnp.int32(0x7FFFFFFF))


def _unmap_f32(m):
    b = jnp.where(m >= 0, m, m ^ jnp.int32(0x7FFFFFFF))
    return jax.lax.bitcast_convert_type(b, jnp.float32)


def _topk_softmax_block(k_ref, x_ref, o_ref, lo_ref, hi_ref, clo_ref, chi_ref):
    x = x_ref[0]
    k = k_ref[pl.program_id(0)]
    R, N = x.shape

    xmin = jnp.min(x, axis=-1, keepdims=True)
    xmax = jnp.max(x, axis=-1, keepdims=True)
    lo_ref[...] = _monotone_i32(jax.lax.bitcast_convert_type(xmin, jnp.int32))
    hi_ref[...] = _monotone_i32(jax.lax.bitcast_convert_type(xmax, jnp.int32))
    clo_ref[...] = jnp.full((R, 1), N, jnp.int32)
    chi_ref[...] = jnp.ones((R, 1), jnp.int32)

    def cond(state):
        i, ndone = state
        return jnp.logical_and(i < _MAX_ITERS, ndone < R)

    def body(state):
        i, _ = state
        lo = lo_ref[...]
        hi = hi_ref[...]
        clo = clo_ref[...]
        chi = chi_ref[...]
        frozen = lo >= hi

        lo_f = lo.astype(jnp.float32)
        hi_f = hi.astype(jnp.float32)
        frac = (clo - k).astype(jnp.float32) / jnp.maximum(
            (clo - chi).astype(jnp.float32), 1.0
        )
        mid_i = jnp.clip(
            (lo_f + (hi_f - lo_f) * frac).astype(jnp.int32), lo + 1, hi
        )
        mid_b = (lo >> 1) + (hi >> 1) + (lo & hi & 1) + ((lo ^ hi) & 1)
        mid = jnp.where(i % 2 == 0, mid_i, mid_b)
        mid = jnp.where(frozen, lo, mid)

        t_f = _unmap_f32(mid)
        cnt = jnp.sum((x >= t_f).astype(jnp.int32), axis=-1, keepdims=True)
        ge = cnt >= k
        eq = cnt == k
        lo_n = jnp.where(eq | ge, mid, lo)
        hi_n = jnp.where(eq, mid, jnp.where(ge, hi, mid - 1))
        lo_ref[...] = jnp.where(frozen, lo, lo_n)
        hi_ref[...] = jnp.where(frozen, hi, hi_n)
        clo_ref[...] = jnp.where(frozen | ~ge, clo, cnt)
        chi_ref[...] = jnp.where(frozen | ge, chi, cnt)

        ndone = jnp.sum((lo_ref[...] >= hi_ref[...]).astype(jnp.int32))
        return i + jnp.int32(1), ndone

    jax.lax.while_loop(cond, body, (jnp.int32(0), jnp.int32(0)))

    t_f = _unmap_f32(lo_ref[...])
    keep = x >= t_f
    e = jnp.where(keep, jnp.exp(x - xmax), 0.0)
    s = jnp.sum(e, axis=-1, keepdims=True)
    o_ref[0] = e / s


@jax.jit
def kernel(attention):
    B, H, M, N = attention.shape
    S = B * H
    x = attention.reshape(S, M, N)
    ks = jnp.tile(
        jnp.array([min(k, N) for k in _K_BY_HEAD], dtype=jnp.int32), B
    )
    R = min(_ROWS_PER_BLOCK, M)
    nb = M // R

    grid_spec = pltpu.PrefetchScalarGridSpec(
        num_scalar_prefetch=1,
        grid=(S, nb),
        in_specs=[
            pl.BlockSpec((1, R, N), lambda s, j, k_ref: (s, j, 0)),
        ],
        out_specs=pl.BlockSpec((1, R, N), lambda s, j, k_ref: (s, j, 0)),
        scratch_shapes=[
            pltpu.VMEM((R, 1), jnp.int32),
            pltpu.VMEM((R, 1), jnp.int32),
            pltpu.VMEM((R, 1), jnp.int32),
            pltpu.VMEM((R, 1), jnp.int32),
        ],
    )
    out = pl.pallas_call(
        _topk_softmax_block,
        grid_spec=grid_spec,
        out_shape=jax.ShapeDtypeStruct((S, M, N), jnp.float32),
        compiler_params=pltpu.CompilerParams(
            dimension_semantics=("parallel", "parallel"),
        ),
    )(ks, x)
    return out.reshape(B, H, M, N)
